# baseline (speedup 1.0000x reference)
import functools

import numpy as _np

import jax
import jax.numpy as jnp
from jax import lax
from jax.experimental import pallas as pl
from jax.experimental.pallas import tpu as pltpu
from jax.experimental.pallas import tpu_sc as plsc

N = 10000
E = 320000
D = 128
DH = 64

NC = 2
NS = 16
NW = NC * NS
EPW = E // NW
C = 200
NCHUNK = EPW // C
CS = 200
NCHUNK_S = EPW // CS
RPS = N // NS

_mesh = plsc.VectorSubcoreMesh(core_axis_name="c", subcore_axis_name="s")


@functools.partial(
    pl.kernel,
    mesh=_mesh,
    out_type=[
        jax.ShapeDtypeStruct((E, D // 2), jnp.float32),
        jax.ShapeDtypeStruct((E, D // 2), jnp.float32),
    ],
    scratch_types=[
        pltpu.VMEM((C,), jnp.int32),
        pltpu.VMEM((C,), jnp.int32),
        pltpu.VMEM((C, D), jnp.float32),
        pltpu.VMEM((C, D), jnp.float32),
        pltpu.VMEM((C, D // 2), jnp.float32),
        pltpu.VMEM((C, D // 2), jnp.float32),
        pltpu.SemaphoreType.DMA,
        pltpu.SemaphoreType.DMA,
    ],
)
def _gather_k(t_hbm, src_hbm, dst_hbm, s_out, ms_out,
              idxs_v, idxd_v, bufd, bufs, sbuf, msbuf, sem1, sem2):
    wid = lax.axis_index("s") * NC + lax.axis_index("c")
    base = wid * EPW

    def chunk(k, carry):
        off = base + k * C
        pltpu.sync_copy(src_hbm.at[pl.ds(off, C)], idxs_v)
        pltpu.sync_copy(dst_hbm.at[pl.ds(off, C)], idxd_v)
        cp1 = pltpu.async_copy(t_hbm.at[idxd_v], bufd, sem1)
        cp2 = pltpu.async_copy(t_hbm.at[idxs_v], bufs, sem2)
        cp1.wait()
        cp2.wait()

        def addrow(r, c2):
            for j in range(2):
                sbuf[r, pl.ds(j * 16, 16)] = bufd[r, pl.ds(j * 16, 16)]
                sbuf[r, pl.ds(32 + j * 16, 16)] = bufs[r, pl.ds(32 + j * 16,
                                                               16)]
            for j in range(4):
                msbuf[r, pl.ds(j * 16, 16)] = bufs[r, pl.ds(64 + j * 16, 16)]
            return c2

        lax.fori_loop(0, C, addrow, 0)
        pltpu.sync_copy(sbuf, s_out.at[pl.ds(off, C)])
        pltpu.sync_copy(msbuf, ms_out.at[pl.ds(off, C)])
        return carry

    lax.fori_loop(0, NCHUNK, chunk, 0)


@functools.partial(
    pl.kernel,
    mesh=_mesh,
    out_type=jax.ShapeDtypeStruct((2 * N, D), jnp.float32),
    scratch_types=[
        pltpu.VMEM((CS,), jnp.int32),
        pltpu.VMEM((CS, D), jnp.float32),
        pltpu.VMEM_SHARED((N, D), jnp.float32),
    ],
)
def _scatter_k(msg_hbm, dst_hbm, out_hbm, idx_v, buf, acc):
    cid = lax.axis_index("c")
    sid = lax.axis_index("s")
    wid = sid * NC + cid

    def zrow(r, carry):
        for j in range(D // 16):
            buf[r, pl.ds(j * 16, 16)] = jnp.zeros((16,), jnp.float32)
        return carry

    lax.fori_loop(0, CS, zrow, 0)
    nzc = N // CS
    for tt in range(-(-nzc // NS)):
        t = tt * NS + sid

        def zcopy(tv=t):
            pltpu.sync_copy(buf, acc.at[pl.ds(tv * CS, CS)])

        pl.when(t < nzc)(zcopy)
    plsc.subcore_barrier()

    base = wid * EPW

    def chunk(k, carry):
        off = base + k * CS
        pltpu.sync_copy(dst_hbm.at[pl.ds(off, CS)], idx_v)
        pltpu.sync_copy(msg_hbm.at[pl.ds(off, CS)], buf)
        pltpu.sync_copy(buf, acc.at[idx_v], add=True)
        return carry

    lax.fori_loop(0, NCHUNK_S, chunk, 0)
    plsc.subcore_barrier()

    for tt in range(-(-nzc // NS)):
        t = tt * NS + sid

        def dcopy(tv=t):
            pltpu.sync_copy(acc.at[pl.ds(tv * CS, CS)],
                            out_hbm.at[pl.ds(cid * N + tv * CS, CS)])

        pl.when(t < nzc)(dcopy)


_NB = 400
_NGRID = N // _NB
_EB = 1600
_EGRID = E // _EB


def _pack_words(x):
    u = lax.bitcast_convert_type(x.astype(jnp.bfloat16), jnp.uint16)
    lo = u[:, :DH].astype(jnp.uint32)
    hi = u[:, DH:].astype(jnp.uint32)
    return lax.bitcast_convert_type(lo | (hi << 16), jnp.float32)


def _node_body(h_ref, wn_ref, a1_ref, a1bias_ref, wet_ref, t_ref):
    hn = jnp.dot(h_ref[...], wn_ref[...], preferred_element_type=jnp.float32)
    p = (jnp.dot(hn, a1_ref[...], preferred_element_type=jnp.float32)
         + a1bias_ref[...])
    m = jnp.dot(hn, wet_ref[...], preferred_element_type=jnp.float32)
    t_ref[...] = jnp.concatenate([_pack_words(p), _pack_words(m)], axis=1)


_node_call = pl.pallas_call(
    _node_body,
    grid=(_NGRID,),
    in_specs=[
        pl.BlockSpec((_NB, D), lambda i: (i, 0)),
        pl.BlockSpec((D, D), lambda i: (0, 0)),
        pl.BlockSpec((D, D), lambda i: (0, 0)),
        pl.BlockSpec((1, D), lambda i: (0, 0)),
        pl.BlockSpec((D, D), lambda i: (0, 0)),
    ],
    out_specs=pl.BlockSpec((_NB, D), lambda i: (i, 0)),
    out_shape=jax.ShapeDtypeStruct((N, D), jnp.float32),
)


def _unpack_cat(words):
    u = lax.bitcast_convert_type(words, jnp.uint32)
    lo = lax.bitcast_convert_type(u << 16, jnp.float32)
    hi = lax.bitcast_convert_type(u & jnp.uint32(0xFFFF0000), jnp.float32)
    return jnp.concatenate([lo, hi], axis=-1)


def _edge_body(s_ref, ms_ref, ea_ref, a2w_ref, a2b_ref, web_ref, msg_ref):
    xw = s_ref[...]
    s = _unpack_cat(xw[:, :D // 4]) + _unpack_cat(xw[:, D // 4:])
    ms = _unpack_cat(ms_ref[...])
    srelu = jnp.maximum(s, 0.0)
    z = jnp.sum(srelu * a2w_ref[...], axis=-1, keepdims=True) + a2b_ref[0, 0]
    att = jax.nn.sigmoid(z)
    ec = (ea_ref[:, 0:1] * web_ref[0:1, :] + ea_ref[:, 1:2] * web_ref[1:2, :])
    msg_ref[...] = att * (ms + ec)


_edge_call = pl.pallas_call(
    _edge_body,
    grid=(_EGRID,),
    in_specs=[
        pl.BlockSpec((_EB, D // 2), lambda i: (i, 0)),
        pl.BlockSpec((_EB, D // 2), lambda i: (i, 0)),
        pl.BlockSpec((_EB, 2), lambda i: (i, 0)),
        pl.BlockSpec((1, DH), lambda i: (0, 0)),
        pl.BlockSpec((1, 1), lambda i: (0, 0)),
        pl.BlockSpec((2, D), lambda i: (0, 0)),
    ],
    out_specs=pl.BlockSpec((_EB, D), lambda i: (i, 0)),
    out_shape=jax.ShapeDtypeStruct((E, D), jnp.float32),
)


def _post_body(p0_ref, p1_ref, hin_ref, b_ref, g_ref, bt_ref, o_ref,
               *, apply_relu):
    t = p0_ref[...] + p1_ref[...] + b_ref[...]
    mu = jnp.mean(t, axis=-1, keepdims=True)
    var = jnp.mean((t - mu) ** 2, axis=-1, keepdims=True)
    y = (t - mu) * lax.rsqrt(var + 1e-5) * g_ref[...] + bt_ref[...]
    if apply_relu:
        y = jnp.maximum(y, 0.0)
    o_ref[...] = y + hin_ref[...]


def _post_call(apply_relu):
    return pl.pallas_call(
        functools.partial(_post_body, apply_relu=apply_relu),
        grid=(_NGRID,),
        in_specs=[
            pl.BlockSpec((_NB, D), lambda i: (i, 0)),
            pl.BlockSpec((_NB, D), lambda i: (i + _NGRID, 0)),
            pl.BlockSpec((_NB, D), lambda i: (i, 0)),
            pl.BlockSpec((1, D), lambda i: (0, 0)),
            pl.BlockSpec((1, D), lambda i: (0, 0)),
            pl.BlockSpec((1, D), lambda i: (0, 0)),
        ],
        out_specs=pl.BlockSpec((_NB, D), lambda i: (i, 0)),
        out_shape=jax.ShapeDtypeStruct((N, D), jnp.float32),
    )


def kernel(x, edge_index, edge_attr,
           Wn0, We0, A1w0, A1b0, A2w0, A2b0, b0, g0, bt0,
           Wn1, We1, A1w1, A1b1, A2w1, A2b1, b1, g1, bt1,
           Wn2, We2, A1w2, A1b2, A2w2, A2b2, b2, g2, bt2):
    src = edge_index[0]
    dst = edge_index[1]
    layers = [
        (Wn0, We0, A1w0, A1b0, A2w0, A2b0, b0, g0, bt0),
        (Wn1, We1, A1w1, A1b1, A2w1, A2b1, b1, g1, bt1),
        (Wn2, We2, A1w2, A1b2, A2w2, A2b2, b2, g2, bt2),
    ]
    catp = _np.concatenate([_np.arange(0, D, 2), _np.arange(1, D, 2)])
    inv = _np.empty(D, dtype=_np.int32)
    inv[0::2] = _np.arange(DH)
    inv[1::2] = _np.arange(DH, D)
    h = x
    for li, (Wn, We, A1w, A1b, A2w, A2b, b, g, bt) in enumerate(layers):
        a1_comb = jnp.concatenate([A1w[:D], A1w[D:]], axis=1)[:, catp]
        a1bias = jnp.concatenate([A1b, jnp.zeros((DH,), A1b.dtype)])[catp]
        wet_cat = We[:D][:, catp]
        t = _node_call(h, Wn, a1_comb, a1bias.reshape(1, D), wet_cat)
        s, ms = _gather_k(t, src, dst)
        a2 = A2w[:, 0]
        a2w_cat = jnp.concatenate([a2[0::2], a2[1::2]]).reshape(1, DH)
        web_cat = jnp.concatenate([We[D:][:, 0::2], We[D:][:, 1::2]], axis=1)
        msg = _edge_call(s, ms, edge_attr, a2w_cat, A2b.reshape(1, 1), web_cat)
        part = jnp.take(_scatter_k(msg, dst), inv, axis=1)
        h = _post_call(li < 2)(part, part, h, b.reshape(1, D), g.reshape(1, D),
                               bt.reshape(1, D))
    return h


# speedup vs baseline: 1.1064x; 1.1064x over previous
import functools

import numpy as _np

import jax
import jax.numpy as jnp
from jax import lax
from jax.experimental import pallas as pl
from jax.experimental.pallas import tpu as pltpu
from jax.experimental.pallas import tpu_sc as plsc

N = 10000
E = 320000
D = 128
DH = 64

NC = 2
NS = 16
NW = NC * NS
EPW = E // NW
C = 80
NCHUNK = EPW // C
CS = 200
NCHUNK_S = EPW // CS
RPS = N // NS

_mesh = plsc.VectorSubcoreMesh(core_axis_name="c", subcore_axis_name="s")


@functools.partial(
    pl.kernel,
    mesh=_mesh,
    out_type=[
        jax.ShapeDtypeStruct((E, D // 2), jnp.float32),
        jax.ShapeDtypeStruct((E, D // 2), jnp.float32),
    ],
    scratch_types=[
        pltpu.VMEM((C,), jnp.int32),
        pltpu.VMEM((C,), jnp.int32),
        pltpu.VMEM((C,), jnp.int32),
        pltpu.VMEM((C,), jnp.int32),
        pltpu.VMEM((C, D), jnp.float32),
        pltpu.VMEM((C, D), jnp.float32),
        pltpu.VMEM((C, D), jnp.float32),
        pltpu.VMEM((C, D), jnp.float32),
        pltpu.VMEM((C, D // 2), jnp.float32),
        pltpu.VMEM((C, D // 2), jnp.float32),
        pltpu.SemaphoreType.DMA,
        pltpu.SemaphoreType.DMA,
        pltpu.SemaphoreType.DMA,
        pltpu.SemaphoreType.DMA,
        pltpu.SemaphoreType.DMA,
        pltpu.SemaphoreType.DMA,
        pltpu.SemaphoreType.DMA,
        pltpu.SemaphoreType.DMA,
        pltpu.SemaphoreType.DMA,
        pltpu.SemaphoreType.DMA,
        pltpu.SemaphoreType.DMA,
        pltpu.SemaphoreType.DMA,
    ],
)
def _gather_k(t_hbm, src_hbm, dst_hbm, s_out, ms_out,
              idxs0, idxs1, idxd0, idxd1, bufd0, bufd1, bufs0, bufs1,
              sbuf, msbuf,
              semd0, semd1, sems0, sems1, semi0, semi1, semj0, semj1,
              semw0, semw1, semv0, semv1):
    wid = lax.axis_index("s") * NC + lax.axis_index("c")
    base = wid * EPW
    idxs = [idxs0, idxs1]
    idxd = [idxd0, idxd1]
    bufd = [bufd0, bufd1]
    bufs = [bufs0, bufs1]
    semd = [semd0, semd1]
    sems_ = [sems0, sems1]
    semi = [semi0, semi1]
    semj = [semj0, semj1]
    semw = [semw0, semw1]
    semv = [semv0, semv1]

    gd = [None, None]
    gs = [None, None]
    ih = [None, None]
    wh = [None, None]

    pltpu.sync_copy(src_hbm.at[pl.ds(base, C)], idxs[0])
    pltpu.sync_copy(dst_hbm.at[pl.ds(base, C)], idxd[0])
    gd[0] = pltpu.async_copy(t_hbm.at[idxd[0]], bufd[0], semd[0])
    gs[0] = pltpu.async_copy(t_hbm.at[idxs[0]], bufs[0], sems_[0])
    if NCHUNK > 1:
        ih[1] = (pltpu.async_copy(src_hbm.at[pl.ds(base + C, C)], idxs[1],
                                  semi[1]),
                 pltpu.async_copy(dst_hbm.at[pl.ds(base + C, C)], idxd[1],
                                  semj[1]))

    for k in range(NCHUNK):
        p = k % 2
        q = 1 - p
        off = base + k * C
        if k + 1 < NCHUNK:
            ih[q][0].wait()
            ih[q][1].wait()
            gd[q] = pltpu.async_copy(t_hbm.at[idxd[q]], bufd[q], semd[q])
            gs[q] = pltpu.async_copy(t_hbm.at[idxs[q]], bufs[q], sems_[q])
        gd[p].wait()
        gs[p].wait()
        if k + 2 < NCHUNK:
            off2 = off + 2 * C
            ih[p] = (pltpu.async_copy(src_hbm.at[pl.ds(off2, C)], idxs[p],
                                      semi[p]),
                     pltpu.async_copy(dst_hbm.at[pl.ds(off2, C)], idxd[p],
                                      semj[p]))
        if wh[0] is not None:
            for h in wh[0]:
                h.wait()
            wh[0] = None

        def addrow(r, c2, _p=p):
            for j in range(2):
                sbuf[r, pl.ds(j * 16, 16)] = bufd[_p][r, pl.ds(j * 16, 16)]
                sbuf[r, pl.ds(32 + j * 16, 16)] = bufs[_p][
                    r, pl.ds(32 + j * 16, 16)]
            for j in range(4):
                msbuf[r, pl.ds(j * 16, 16)] = bufs[_p][r,
                                                       pl.ds(64 + j * 16, 16)]
            return c2

        lax.fori_loop(0, C, addrow, 0)
        wh[0] = (
            pltpu.async_copy(sbuf, s_out.at[pl.ds(off, C)], semw[p]),
            pltpu.async_copy(msbuf, ms_out.at[pl.ds(off, C)], semv[p]),
        )

    if wh[0] is not None:
        for h in wh[0]:
            h.wait()


@functools.partial(
    pl.kernel,
    mesh=_mesh,
    out_type=jax.ShapeDtypeStruct((2 * N, D), jnp.float32),
    scratch_types=[
        pltpu.VMEM((CS,), jnp.int32),
        pltpu.VMEM((CS, D), jnp.float32),
        pltpu.VMEM_SHARED((N, D), jnp.float32),
    ],
)
def _scatter_k(msg_hbm, dst_hbm, out_hbm, idx_v, buf, acc):
    cid = lax.axis_index("c")
    sid = lax.axis_index("s")
    wid = sid * NC + cid

    def zrow(r, carry):
        for j in range(D // 16):
            buf[r, pl.ds(j * 16, 16)] = jnp.zeros((16,), jnp.float32)
        return carry

    lax.fori_loop(0, CS, zrow, 0)
    nzc = N // CS
    for tt in range(-(-nzc // NS)):
        t = tt * NS + sid

        def zcopy(tv=t):
            pltpu.sync_copy(buf, acc.at[pl.ds(tv * CS, CS)])

        pl.when(t < nzc)(zcopy)
    plsc.subcore_barrier()

    base = wid * EPW

    def chunk(k, carry):
        off = base + k * CS
        pltpu.sync_copy(dst_hbm.at[pl.ds(off, CS)], idx_v)
        pltpu.sync_copy(msg_hbm.at[pl.ds(off, CS)], buf)
        pltpu.sync_copy(buf, acc.at[idx_v], add=True)
        return carry

    lax.fori_loop(0, NCHUNK_S, chunk, 0)
    plsc.subcore_barrier()

    for tt in range(-(-nzc // NS)):
        t = tt * NS + sid

        def dcopy(tv=t):
            pltpu.sync_copy(acc.at[pl.ds(tv * CS, CS)],
                            out_hbm.at[pl.ds(cid * N + tv * CS, CS)])

        pl.when(t < nzc)(dcopy)


_NB = 400
_NGRID = N // _NB
_EB = 1600
_EGRID = E // _EB


def _pack_words(x):
    u = lax.bitcast_convert_type(x.astype(jnp.bfloat16), jnp.uint16)
    lo = u[:, :DH].astype(jnp.uint32)
    hi = u[:, DH:].astype(jnp.uint32)
    return lax.bitcast_convert_type(lo | (hi << 16), jnp.float32)


def _node_body(h_ref, wn_ref, a1_ref, a1bias_ref, wet_ref, t_ref):
    hn = jnp.dot(h_ref[...], wn_ref[...], preferred_element_type=jnp.float32)
    p = (jnp.dot(hn, a1_ref[...], preferred_element_type=jnp.float32)
         + a1bias_ref[...])
    m = jnp.dot(hn, wet_ref[...], preferred_element_type=jnp.float32)
    t_ref[...] = jnp.concatenate([_pack_words(p), _pack_words(m)], axis=1)


_node_call = pl.pallas_call(
    _node_body,
    grid=(_NGRID,),
    in_specs=[
        pl.BlockSpec((_NB, D), lambda i: (i, 0)),
        pl.BlockSpec((D, D), lambda i: (0, 0)),
        pl.BlockSpec((D, D), lambda i: (0, 0)),
        pl.BlockSpec((1, D), lambda i: (0, 0)),
        pl.BlockSpec((D, D), lambda i: (0, 0)),
    ],
    out_specs=pl.BlockSpec((_NB, D), lambda i: (i, 0)),
    out_shape=jax.ShapeDtypeStruct((N, D), jnp.float32),
)


def _unpack_cat(words):
    u = lax.bitcast_convert_type(words, jnp.uint32)
    lo = lax.bitcast_convert_type(u << 16, jnp.float32)
    hi = lax.bitcast_convert_type(u & jnp.uint32(0xFFFF0000), jnp.float32)
    return jnp.concatenate([lo, hi], axis=-1)


def _edge_body(s_ref, ms_ref, ea_ref, a2w_ref, a2b_ref, web_ref, msg_ref):
    xw = s_ref[...]
    s = _unpack_cat(xw[:, :D // 4]) + _unpack_cat(xw[:, D // 4:])
    ms = _unpack_cat(ms_ref[...])
    srelu = jnp.maximum(s, 0.0)
    z = jnp.sum(srelu * a2w_ref[...], axis=-1, keepdims=True) + a2b_ref[0, 0]
    att = jax.nn.sigmoid(z)
    ec = (ea_ref[:, 0:1] * web_ref[0:1, :] + ea_ref[:, 1:2] * web_ref[1:2, :])
    msg_ref[...] = att * (ms + ec)


_edge_call = pl.pallas_call(
    _edge_body,
    grid=(_EGRID,),
    in_specs=[
        pl.BlockSpec((_EB, D // 2), lambda i: (i, 0)),
        pl.BlockSpec((_EB, D // 2), lambda i: (i, 0)),
        pl.BlockSpec((_EB, 2), lambda i: (i, 0)),
        pl.BlockSpec((1, DH), lambda i: (0, 0)),
        pl.BlockSpec((1, 1), lambda i: (0, 0)),
        pl.BlockSpec((2, D), lambda i: (0, 0)),
    ],
    out_specs=pl.BlockSpec((_EB, D), lambda i: (i, 0)),
    out_shape=jax.ShapeDtypeStruct((E, D), jnp.float32),
)


def _post_body(p0_ref, p1_ref, hin_ref, b_ref, g_ref, bt_ref, o_ref,
               *, apply_relu):
    t = p0_ref[...] + p1_ref[...] + b_ref[...]
    mu = jnp.mean(t, axis=-1, keepdims=True)
    var = jnp.mean((t - mu) ** 2, axis=-1, keepdims=True)
    y = (t - mu) * lax.rsqrt(var + 1e-5) * g_ref[...] + bt_ref[...]
    if apply_relu:
        y = jnp.maximum(y, 0.0)
    o_ref[...] = y + hin_ref[...]


def _post_call(apply_relu):
    return pl.pallas_call(
        functools.partial(_post_body, apply_relu=apply_relu),
        grid=(_NGRID,),
        in_specs=[
            pl.BlockSpec((_NB, D), lambda i: (i, 0)),
            pl.BlockSpec((_NB, D), lambda i: (i + _NGRID, 0)),
            pl.BlockSpec((_NB, D), lambda i: (i, 0)),
            pl.BlockSpec((1, D), lambda i: (0, 0)),
            pl.BlockSpec((1, D), lambda i: (0, 0)),
            pl.BlockSpec((1, D), lambda i: (0, 0)),
        ],
        out_specs=pl.BlockSpec((_NB, D), lambda i: (i, 0)),
        out_shape=jax.ShapeDtypeStruct((N, D), jnp.float32),
    )


def kernel(x, edge_index, edge_attr,
           Wn0, We0, A1w0, A1b0, A2w0, A2b0, b0, g0, bt0,
           Wn1, We1, A1w1, A1b1, A2w1, A2b1, b1, g1, bt1,
           Wn2, We2, A1w2, A1b2, A2w2, A2b2, b2, g2, bt2):
    src = edge_index[0]
    dst = edge_index[1]
    layers = [
        (Wn0, We0, A1w0, A1b0, A2w0, A2b0, b0, g0, bt0),
        (Wn1, We1, A1w1, A1b1, A2w1, A2b1, b1, g1, bt1),
        (Wn2, We2, A1w2, A1b2, A2w2, A2b2, b2, g2, bt2),
    ]
    catp = _np.concatenate([_np.arange(0, D, 2), _np.arange(1, D, 2)])
    inv = _np.empty(D, dtype=_np.int32)
    inv[0::2] = _np.arange(DH)
    inv[1::2] = _np.arange(DH, D)
    h = x
    for li, (Wn, We, A1w, A1b, A2w, A2b, b, g, bt) in enumerate(layers):
        a1_comb = jnp.concatenate([A1w[:D], A1w[D:]], axis=1)[:, catp]
        a1bias = jnp.concatenate([A1b, jnp.zeros((DH,), A1b.dtype)])[catp]
        wet_cat = We[:D][:, catp]
        t = _node_call(h, Wn, a1_comb, a1bias.reshape(1, D), wet_cat)
        s, ms = _gather_k(t, src, dst)
        a2 = A2w[:, 0]
        a2w_cat = jnp.concatenate([a2[0::2], a2[1::2]]).reshape(1, DH)
        web_cat = jnp.concatenate([We[D:][:, 0::2], We[D:][:, 1::2]], axis=1)
        msg = _edge_call(s, ms, edge_attr, a2w_cat, A2b.reshape(1, 1), web_cat)
        part = jnp.take(_scatter_k(msg, dst), inv, axis=1)
        h = _post_call(li < 2)(part, part, h, b.reshape(1, D), g.reshape(1, D),
                               bt.reshape(1, D))
    return h


# speedup vs baseline: 1.1351x; 1.0259x over previous
import functools

import numpy as _np

import jax
import jax.numpy as jnp
from jax import lax
from jax.experimental import pallas as pl
from jax.experimental.pallas import tpu as pltpu
from jax.experimental.pallas import tpu_sc as plsc

N = 10000
E = 320000
D = 128
DH = 64

NC = 2
NS = 16
NW = NC * NS
EPW = E // NW
C = 80
NCHUNK = EPW // C
CS = 80
NCHUNK_S = EPW // CS
RPS = N // NS

_mesh = plsc.VectorSubcoreMesh(core_axis_name="c", subcore_axis_name="s")


@functools.partial(
    pl.kernel,
    mesh=_mesh,
    out_type=[
        jax.ShapeDtypeStruct((E, D // 2), jnp.float32),
        jax.ShapeDtypeStruct((E, D // 2), jnp.float32),
    ],
    scratch_types=[
        pltpu.VMEM((C,), jnp.int32),
        pltpu.VMEM((C,), jnp.int32),
        pltpu.VMEM((C,), jnp.int32),
        pltpu.VMEM((C,), jnp.int32),
        pltpu.VMEM((C, D), jnp.float32),
        pltpu.VMEM((C, D), jnp.float32),
        pltpu.VMEM((C, D), jnp.float32),
        pltpu.VMEM((C, D), jnp.float32),
        pltpu.VMEM((C, D // 2), jnp.float32),
        pltpu.VMEM((C, D // 2), jnp.float32),
        pltpu.SemaphoreType.DMA,
        pltpu.SemaphoreType.DMA,
        pltpu.SemaphoreType.DMA,
        pltpu.SemaphoreType.DMA,
        pltpu.SemaphoreType.DMA,
        pltpu.SemaphoreType.DMA,
        pltpu.SemaphoreType.DMA,
        pltpu.SemaphoreType.DMA,
        pltpu.SemaphoreType.DMA,
        pltpu.SemaphoreType.DMA,
        pltpu.SemaphoreType.DMA,
        pltpu.SemaphoreType.DMA,
    ],
)
def _gather_k(t_hbm, src_hbm, dst_hbm, s_out, ms_out,
              idxs0, idxs1, idxd0, idxd1, bufd0, bufd1, bufs0, bufs1,
              sbuf, msbuf,
              semd0, semd1, sems0, sems1, semi0, semi1, semj0, semj1,
              semw0, semw1, semv0, semv1):
    wid = lax.axis_index("s") * NC + lax.axis_index("c")
    base = wid * EPW
    idxs = [idxs0, idxs1]
    idxd = [idxd0, idxd1]
    bufd = [bufd0, bufd1]
    bufs = [bufs0, bufs1]
    semd = [semd0, semd1]
    sems_ = [sems0, sems1]
    semi = [semi0, semi1]
    semj = [semj0, semj1]
    semw = [semw0, semw1]
    semv = [semv0, semv1]

    gd = [None, None]
    gs = [None, None]
    ih = [None, None]
    wh = [None, None]

    pltpu.sync_copy(src_hbm.at[pl.ds(base, C)], idxs[0])
    pltpu.sync_copy(dst_hbm.at[pl.ds(base, C)], idxd[0])
    gd[0] = pltpu.async_copy(t_hbm.at[idxd[0]], bufd[0], semd[0])
    gs[0] = pltpu.async_copy(t_hbm.at[idxs[0]], bufs[0], sems_[0])
    if NCHUNK > 1:
        ih[1] = (pltpu.async_copy(src_hbm.at[pl.ds(base + C, C)], idxs[1],
                                  semi[1]),
                 pltpu.async_copy(dst_hbm.at[pl.ds(base + C, C)], idxd[1],
                                  semj[1]))

    for k in range(NCHUNK):
        p = k % 2
        q = 1 - p
        off = base + k * C
        if k + 1 < NCHUNK:
            ih[q][0].wait()
            ih[q][1].wait()
            gd[q] = pltpu.async_copy(t_hbm.at[idxd[q]], bufd[q], semd[q])
            gs[q] = pltpu.async_copy(t_hbm.at[idxs[q]], bufs[q], sems_[q])
        gd[p].wait()
        gs[p].wait()
        if k + 2 < NCHUNK:
            off2 = off + 2 * C
            ih[p] = (pltpu.async_copy(src_hbm.at[pl.ds(off2, C)], idxs[p],
                                      semi[p]),
                     pltpu.async_copy(dst_hbm.at[pl.ds(off2, C)], idxd[p],
                                      semj[p]))
        if wh[0] is not None:
            for h in wh[0]:
                h.wait()
            wh[0] = None

        def addrow(r, c2, _p=p):
            for j in range(2):
                sbuf[r, pl.ds(j * 16, 16)] = bufd[_p][r, pl.ds(j * 16, 16)]
                sbuf[r, pl.ds(32 + j * 16, 16)] = bufs[_p][
                    r, pl.ds(32 + j * 16, 16)]
            for j in range(4):
                msbuf[r, pl.ds(j * 16, 16)] = bufs[_p][r,
                                                       pl.ds(64 + j * 16, 16)]
            return c2

        lax.fori_loop(0, C, addrow, 0)
        wh[0] = (
            pltpu.async_copy(sbuf, s_out.at[pl.ds(off, C)], semw[p]),
            pltpu.async_copy(msbuf, ms_out.at[pl.ds(off, C)], semv[p]),
        )

    if wh[0] is not None:
        for h in wh[0]:
            h.wait()


@functools.partial(
    pl.kernel,
    mesh=_mesh,
    out_type=jax.ShapeDtypeStruct((2 * N, D), jnp.float32),
    scratch_types=[
        pltpu.VMEM((CS,), jnp.int32),
        pltpu.VMEM((CS,), jnp.int32),
        pltpu.VMEM((CS, D), jnp.float32),
        pltpu.VMEM((CS, D), jnp.float32),
        pltpu.VMEM_SHARED((N, D), jnp.float32),
        pltpu.SemaphoreType.DMA,
        pltpu.SemaphoreType.DMA,
        pltpu.SemaphoreType.DMA,
        pltpu.SemaphoreType.DMA,
    ],
)
def _scatter_k(msg_hbm, dst_hbm, out_hbm, idx0, idx1, buf0, buf1, acc,
               semm0, semm1, semi0, semi1):
    cid = lax.axis_index("c")
    sid = lax.axis_index("s")
    wid = sid * NC + cid
    idx = [idx0, idx1]
    buf = [buf0, buf1]
    semm = [semm0, semm1]
    semi = [semi0, semi1]

    def zrow(r, carry):
        for j in range(D // 16):
            buf0[r, pl.ds(j * 16, 16)] = jnp.zeros((16,), jnp.float32)
        return carry

    lax.fori_loop(0, CS, zrow, 0)
    nzc = N // CS
    for tt in range(-(-nzc // NS)):
        t = tt * NS + sid

        def zcopy(tv=t):
            pltpu.sync_copy(buf0, acc.at[pl.ds(tv * CS, CS)])

        pl.when(t < nzc)(zcopy)
    plsc.subcore_barrier()

    base = wid * EPW
    lh = [None, None]
    lh[0] = (pltpu.async_copy(msg_hbm.at[pl.ds(base, CS)], buf[0], semm[0]),
             pltpu.async_copy(dst_hbm.at[pl.ds(base, CS)], idx[0], semi[0]))
    for k in range(NCHUNK_S):
        p = k % 2
        q = 1 - p
        off = base + k * CS
        lh[p][0].wait()
        lh[p][1].wait()
        if k + 1 < NCHUNK_S:
            lh[q] = (pltpu.async_copy(msg_hbm.at[pl.ds(off + CS, CS)],
                                      buf[q], semm[q]),
                     pltpu.async_copy(dst_hbm.at[pl.ds(off + CS, CS)],
                                     idx[q], semi[q]))
        pltpu.sync_copy(buf[p], acc.at[idx[p]], add=True)
    plsc.subcore_barrier()

    for tt in range(-(-nzc // NS)):
        t = tt * NS + sid

        def dcopy(tv=t):
            pltpu.sync_copy(acc.at[pl.ds(tv * CS, CS)],
                            out_hbm.at[pl.ds(cid * N + tv * CS, CS)])

        pl.when(t < nzc)(dcopy)


_NB = 400
_NGRID = N // _NB
_EB = 1600
_EGRID = E // _EB


def _pack_words(x):
    u = lax.bitcast_convert_type(x.astype(jnp.bfloat16), jnp.uint16)
    lo = u[:, :DH].astype(jnp.uint32)
    hi = u[:, DH:].astype(jnp.uint32)
    return lax.bitcast_convert_type(lo | (hi << 16), jnp.float32)


def _node_body(h_ref, wn_ref, a1_ref, a1bias_ref, wet_ref, t_ref):
    hn = jnp.dot(h_ref[...], wn_ref[...], preferred_element_type=jnp.float32)
    p = (jnp.dot(hn, a1_ref[...], preferred_element_type=jnp.float32)
         + a1bias_ref[...])
    m = jnp.dot(hn, wet_ref[...], preferred_element_type=jnp.float32)
    t_ref[...] = jnp.concatenate([_pack_words(p), _pack_words(m)], axis=1)


_node_call = pl.pallas_call(
    _node_body,
    grid=(_NGRID,),
    in_specs=[
        pl.BlockSpec((_NB, D), lambda i: (i, 0)),
        pl.BlockSpec((D, D), lambda i: (0, 0)),
        pl.BlockSpec((D, D), lambda i: (0, 0)),
        pl.BlockSpec((1, D), lambda i: (0, 0)),
        pl.BlockSpec((D, D), lambda i: (0, 0)),
    ],
    out_specs=pl.BlockSpec((_NB, D), lambda i: (i, 0)),
    out_shape=jax.ShapeDtypeStruct((N, D), jnp.float32),
)


def _unpack_cat(words):
    u = lax.bitcast_convert_type(words, jnp.uint32)
    lo = lax.bitcast_convert_type(u << 16, jnp.float32)
    hi = lax.bitcast_convert_type(u & jnp.uint32(0xFFFF0000), jnp.float32)
    return jnp.concatenate([lo, hi], axis=-1)


def _edge_body(s_ref, ms_ref, ea_ref, a2w_ref, a2b_ref, web_ref, msg_ref):
    xw = s_ref[...]
    s = _unpack_cat(xw[:, :D // 4]) + _unpack_cat(xw[:, D // 4:])
    ms = _unpack_cat(ms_ref[...])
    srelu = jnp.maximum(s, 0.0)
    z = jnp.sum(srelu * a2w_ref[...], axis=-1, keepdims=True) + a2b_ref[0, 0]
    att = jax.nn.sigmoid(z)
    ec = (ea_ref[:, 0:1] * web_ref[0:1, :] + ea_ref[:, 1:2] * web_ref[1:2, :])
    msg_ref[...] = att * (ms + ec)


_edge_call = pl.pallas_call(
    _edge_body,
    grid=(_EGRID,),
    in_specs=[
        pl.BlockSpec((_EB, D // 2), lambda i: (i, 0)),
        pl.BlockSpec((_EB, D // 2), lambda i: (i, 0)),
        pl.BlockSpec((_EB, 2), lambda i: (i, 0)),
        pl.BlockSpec((1, DH), lambda i: (0, 0)),
        pl.BlockSpec((1, 1), lambda i: (0, 0)),
        pl.BlockSpec((2, D), lambda i: (0, 0)),
    ],
    out_specs=pl.BlockSpec((_EB, D), lambda i: (i, 0)),
    out_shape=jax.ShapeDtypeStruct((E, D), jnp.float32),
)


def _post_body(p0_ref, p1_ref, hin_ref, b_ref, g_ref, bt_ref, o_ref,
               *, apply_relu):
    t = p0_ref[...] + p1_ref[...] + b_ref[...]
    mu = jnp.mean(t, axis=-1, keepdims=True)
    var = jnp.mean((t - mu) ** 2, axis=-1, keepdims=True)
    y = (t - mu) * lax.rsqrt(var + 1e-5) * g_ref[...] + bt_ref[...]
    if apply_relu:
        y = jnp.maximum(y, 0.0)
    o_ref[...] = y + hin_ref[...]


def _post_call(apply_relu):
    return pl.pallas_call(
        functools.partial(_post_body, apply_relu=apply_relu),
        grid=(_NGRID,),
        in_specs=[
            pl.BlockSpec((_NB, D), lambda i: (i, 0)),
            pl.BlockSpec((_NB, D), lambda i: (i + _NGRID, 0)),
            pl.BlockSpec((_NB, D), lambda i: (i, 0)),
            pl.BlockSpec((1, D), lambda i: (0, 0)),
            pl.BlockSpec((1, D), lambda i: (0, 0)),
            pl.BlockSpec((1, D), lambda i: (0, 0)),
        ],
        out_specs=pl.BlockSpec((_NB, D), lambda i: (i, 0)),
        out_shape=jax.ShapeDtypeStruct((N, D), jnp.float32),
    )


def kernel(x, edge_index, edge_attr,
           Wn0, We0, A1w0, A1b0, A2w0, A2b0, b0, g0, bt0,
           Wn1, We1, A1w1, A1b1, A2w1, A2b1, b1, g1, bt1,
           Wn2, We2, A1w2, A1b2, A2w2, A2b2, b2, g2, bt2):
    src = edge_index[0]
    dst = edge_index[1]
    layers = [
        (Wn0, We0, A1w0, A1b0, A2w0, A2b0, b0, g0, bt0),
        (Wn1, We1, A1w1, A1b1, A2w1, A2b1, b1, g1, bt1),
        (Wn2, We2, A1w2, A1b2, A2w2, A2b2, b2, g2, bt2),
    ]
    catp = _np.concatenate([_np.arange(0, D, 2), _np.arange(1, D, 2)])
    inv = _np.empty(D, dtype=_np.int32)
    inv[0::2] = _np.arange(DH)
    inv[1::2] = _np.arange(DH, D)
    h = x
    for li, (Wn, We, A1w, A1b, A2w, A2b, b, g, bt) in enumerate(layers):
        a1_comb = jnp.concatenate([A1w[:D], A1w[D:]], axis=1)[:, catp]
        a1bias = jnp.concatenate([A1b, jnp.zeros((DH,), A1b.dtype)])[catp]
        wet_cat = We[:D][:, catp]
        t = _node_call(h, Wn, a1_comb, a1bias.reshape(1, D), wet_cat)
        s, ms = _gather_k(t, src, dst)
        a2 = A2w[:, 0]
        a2w_cat = jnp.concatenate([a2[0::2], a2[1::2]]).reshape(1, DH)
        web_cat = jnp.concatenate([We[D:][:, 0::2], We[D:][:, 1::2]], axis=1)
        msg = _edge_call(s, ms, edge_attr, a2w_cat, A2b.reshape(1, 1), web_cat)
        part = jnp.take(_scatter_k(msg, dst), inv, axis=1)
        h = _post_call(li < 2)(part, part, h, b.reshape(1, D), g.reshape(1, D),
                               bt.reshape(1, D))
    return h


# speedup vs baseline: 1.1864x; 1.0452x over previous
import functools

import numpy as _np

import jax
import jax.numpy as jnp
from jax import lax
from jax.experimental import pallas as pl
from jax.experimental.pallas import tpu as pltpu
from jax.experimental.pallas import tpu_sc as plsc

N = 10000
E = 320000
D = 128
DH = 64

NC = 2
NS = 16
NW = NC * NS
EH = E // 2
EPW = EH // NW
C = 40
NCHUNK = EPW // C
CS = 40
NCHUNK_S = EPW // CS
RPS = N // NS

_mesh = plsc.VectorSubcoreMesh(core_axis_name="c", subcore_axis_name="s")


def _make_gather(hoff):
  @functools.partial(
    pl.kernel,
    mesh=_mesh,
    out_type=[
        jax.ShapeDtypeStruct((EH, D // 2), jnp.float32),
        jax.ShapeDtypeStruct((EH, D // 2), jnp.float32),
    ],
    scratch_types=[
        pltpu.VMEM((C,), jnp.int32),
        pltpu.VMEM((C,), jnp.int32),
        pltpu.VMEM((C,), jnp.int32),
        pltpu.VMEM((C,), jnp.int32),
        pltpu.VMEM((C, D), jnp.float32),
        pltpu.VMEM((C, D), jnp.float32),
        pltpu.VMEM((C, D), jnp.float32),
        pltpu.VMEM((C, D), jnp.float32),
        pltpu.VMEM((C, D // 2), jnp.float32),
        pltpu.VMEM((C, D // 2), jnp.float32),
        pltpu.SemaphoreType.DMA,
        pltpu.SemaphoreType.DMA,
        pltpu.SemaphoreType.DMA,
        pltpu.SemaphoreType.DMA,
        pltpu.SemaphoreType.DMA,
        pltpu.SemaphoreType.DMA,
        pltpu.SemaphoreType.DMA,
        pltpu.SemaphoreType.DMA,
        pltpu.SemaphoreType.DMA,
        pltpu.SemaphoreType.DMA,
        pltpu.SemaphoreType.DMA,
        pltpu.SemaphoreType.DMA,
    ],
  )
  def _gather_k(t_hbm, src_hbm, dst_hbm, s_out, ms_out,
              idxs0, idxs1, idxd0, idxd1, bufd0, bufd1, bufs0, bufs1,
              sbuf, msbuf,
              semd0, semd1, sems0, sems1, semi0, semi1, semj0, semj1,
              semw0, semw1, semv0, semv1):
    wid = lax.axis_index("s") * NC + lax.axis_index("c")
    obase = wid * EPW
    base = hoff + obase
    idxs = [idxs0, idxs1]
    idxd = [idxd0, idxd1]
    bufd = [bufd0, bufd1]
    bufs = [bufs0, bufs1]
    semd = [semd0, semd1]
    sems_ = [sems0, sems1]
    semi = [semi0, semi1]
    semj = [semj0, semj1]
    semw = [semw0, semw1]
    semv = [semv0, semv1]

    gd = [None, None]
    gs = [None, None]
    ih = [None, None]
    wh = [None, None]

    pltpu.sync_copy(src_hbm.at[pl.ds(base, C)], idxs[0])
    pltpu.sync_copy(dst_hbm.at[pl.ds(base, C)], idxd[0])
    gd[0] = pltpu.async_copy(t_hbm.at[idxd[0]], bufd[0], semd[0])
    gs[0] = pltpu.async_copy(t_hbm.at[idxs[0]], bufs[0], sems_[0])
    if NCHUNK > 1:
        ih[1] = (pltpu.async_copy(src_hbm.at[pl.ds(base + C, C)], idxs[1],
                                  semi[1]),
                 pltpu.async_copy(dst_hbm.at[pl.ds(base + C, C)], idxd[1],
                                  semj[1]))

    for k in range(NCHUNK):
        p = k % 2
        q = 1 - p
        off = base + k * C
        if k + 1 < NCHUNK:
            ih[q][0].wait()
            ih[q][1].wait()
            gd[q] = pltpu.async_copy(t_hbm.at[idxd[q]], bufd[q], semd[q])
            gs[q] = pltpu.async_copy(t_hbm.at[idxs[q]], bufs[q], sems_[q])
        gd[p].wait()
        gs[p].wait()
        if k + 2 < NCHUNK:
            off2 = off + 2 * C
            ih[p] = (pltpu.async_copy(src_hbm.at[pl.ds(off2, C)], idxs[p],
                                      semi[p]),
                     pltpu.async_copy(dst_hbm.at[pl.ds(off2, C)], idxd[p],
                                      semj[p]))
        if wh[0] is not None:
            for h in wh[0]:
                h.wait()
            wh[0] = None

        def addrow(r, c2, _p=p):
            for j in range(2):
                sbuf[r, pl.ds(j * 16, 16)] = bufd[_p][r, pl.ds(j * 16, 16)]
                sbuf[r, pl.ds(32 + j * 16, 16)] = bufs[_p][
                    r, pl.ds(32 + j * 16, 16)]
            for j in range(4):
                msbuf[r, pl.ds(j * 16, 16)] = bufs[_p][r,
                                                       pl.ds(64 + j * 16, 16)]
            return c2

        lax.fori_loop(0, C, addrow, 0)
        wh[0] = (
            pltpu.async_copy(sbuf, s_out.at[pl.ds(obase + k * C, C)],
                             semw[p]),
            pltpu.async_copy(msbuf, ms_out.at[pl.ds(obase + k * C, C)],
                             semv[p]),
        )

    if wh[0] is not None:
        for h in wh[0]:
            h.wait()

  return _gather_k


_gather_A = _make_gather(0)
_gather_B = _make_gather(EH)


def _make_scatter(hoff):
  @functools.partial(
    pl.kernel,
    mesh=_mesh,
    out_type=jax.ShapeDtypeStruct((2 * N, D), jnp.float32),
    scratch_types=[
        pltpu.VMEM((CS,), jnp.int32),
        pltpu.VMEM((CS,), jnp.int32),
        pltpu.VMEM((CS, D), jnp.float32),
        pltpu.VMEM((CS, D), jnp.float32),
        pltpu.VMEM_SHARED((N, D), jnp.float32),
        pltpu.SemaphoreType.DMA,
        pltpu.SemaphoreType.DMA,
        pltpu.SemaphoreType.DMA,
        pltpu.SemaphoreType.DMA,
    ],
  )
  def _scatter_k(msg_hbm, dst_hbm, out_hbm, idx0, idx1, buf0, buf1, acc,
               semm0, semm1, semi0, semi1):
    cid = lax.axis_index("c")
    sid = lax.axis_index("s")
    wid = sid * NC + cid
    idx = [idx0, idx1]
    buf = [buf0, buf1]
    semm = [semm0, semm1]
    semi = [semi0, semi1]

    def zrow(r, carry):
        for j in range(D // 16):
            buf0[r, pl.ds(j * 16, 16)] = jnp.zeros((16,), jnp.float32)
        return carry

    lax.fori_loop(0, CS, zrow, 0)
    nzc = N // CS
    for tt in range(-(-nzc // NS)):
        t = tt * NS + sid

        def zcopy(tv=t):
            pltpu.sync_copy(buf0, acc.at[pl.ds(tv * CS, CS)])

        pl.when(t < nzc)(zcopy)
    plsc.subcore_barrier()

    obase = wid * EPW
    gbase = hoff + obase
    lh = [None, None]
    lh[0] = (pltpu.async_copy(msg_hbm.at[pl.ds(obase, CS)], buf[0], semm[0]),
             pltpu.async_copy(dst_hbm.at[pl.ds(gbase, CS)], idx[0], semi[0]))
    for k in range(NCHUNK_S):
        p = k % 2
        q = 1 - p
        lh[p][0].wait()
        lh[p][1].wait()
        if k + 1 < NCHUNK_S:
            lh[q] = (pltpu.async_copy(
                         msg_hbm.at[pl.ds(obase + (k + 1) * CS, CS)],
                         buf[q], semm[q]),
                     pltpu.async_copy(
                         dst_hbm.at[pl.ds(gbase + (k + 1) * CS, CS)],
                         idx[q], semi[q]))
        pltpu.sync_copy(buf[p], acc.at[idx[p]], add=True)
    plsc.subcore_barrier()

    for tt in range(-(-nzc // NS)):
        t = tt * NS + sid

        def dcopy(tv=t):
            pltpu.sync_copy(acc.at[pl.ds(tv * CS, CS)],
                            out_hbm.at[pl.ds(cid * N + tv * CS, CS)])

        pl.when(t < nzc)(dcopy)

  return _scatter_k


_scatter_A = _make_scatter(0)
_scatter_B = _make_scatter(EH)


_NB = 400
_NGRID = N // _NB
_EB = 1600
_EGRID = EH // _EB


def _pack_words(x):
    u = lax.bitcast_convert_type(x.astype(jnp.bfloat16), jnp.uint16)
    lo = u[:, :DH].astype(jnp.uint32)
    hi = u[:, DH:].astype(jnp.uint32)
    return lax.bitcast_convert_type(lo | (hi << 16), jnp.float32)


def _node_body(h_ref, wn_ref, a1_ref, a1bias_ref, wet_ref, t_ref):
    hn = jnp.dot(h_ref[...], wn_ref[...], preferred_element_type=jnp.float32)
    p = (jnp.dot(hn, a1_ref[...], preferred_element_type=jnp.float32)
         + a1bias_ref[...])
    m = jnp.dot(hn, wet_ref[...], preferred_element_type=jnp.float32)
    t_ref[...] = jnp.concatenate([_pack_words(p), _pack_words(m)], axis=1)


_node_call = pl.pallas_call(
    _node_body,
    grid=(_NGRID,),
    in_specs=[
        pl.BlockSpec((_NB, D), lambda i: (i, 0)),
        pl.BlockSpec((D, D), lambda i: (0, 0)),
        pl.BlockSpec((D, D), lambda i: (0, 0)),
        pl.BlockSpec((1, D), lambda i: (0, 0)),
        pl.BlockSpec((D, D), lambda i: (0, 0)),
    ],
    out_specs=pl.BlockSpec((_NB, D), lambda i: (i, 0)),
    out_shape=jax.ShapeDtypeStruct((N, D), jnp.float32),
)


def _unpack_cat(words):
    u = lax.bitcast_convert_type(words, jnp.uint32)
    lo = lax.bitcast_convert_type(u << 16, jnp.float32)
    hi = lax.bitcast_convert_type(u & jnp.uint32(0xFFFF0000), jnp.float32)
    return jnp.concatenate([lo, hi], axis=-1)


def _edge_body(s_ref, ms_ref, ea_ref, a2w_ref, a2b_ref, web_ref, msg_ref):
    xw = s_ref[...]
    s = _unpack_cat(xw[:, :D // 4]) + _unpack_cat(xw[:, D // 4:])
    ms = _unpack_cat(ms_ref[...])
    srelu = jnp.maximum(s, 0.0)
    z = jnp.sum(srelu * a2w_ref[...], axis=-1, keepdims=True) + a2b_ref[0, 0]
    att = jax.nn.sigmoid(z)
    ec = (ea_ref[:, 0:1] * web_ref[0:1, :] + ea_ref[:, 1:2] * web_ref[1:2, :])
    msg_ref[...] = att * (ms + ec)


_edge_call = pl.pallas_call(
    _edge_body,
    grid=(_EGRID,),
    in_specs=[
        pl.BlockSpec((_EB, D // 2), lambda i: (i, 0)),
        pl.BlockSpec((_EB, D // 2), lambda i: (i, 0)),
        pl.BlockSpec((_EB, 2), lambda i: (i, 0)),
        pl.BlockSpec((1, DH), lambda i: (0, 0)),
        pl.BlockSpec((1, 1), lambda i: (0, 0)),
        pl.BlockSpec((2, D), lambda i: (0, 0)),
    ],
    out_specs=pl.BlockSpec((_EB, D), lambda i: (i, 0)),
    out_shape=jax.ShapeDtypeStruct((EH, D), jnp.float32),
)


def _post_body(p0_ref, p1_ref, p2_ref, p3_ref, hin_ref, b_ref, g_ref,
               bt_ref, o_ref, *, apply_relu):
    t = (p0_ref[...] + p1_ref[...] + p2_ref[...] + p3_ref[...]
         + b_ref[...])
    mu = jnp.mean(t, axis=-1, keepdims=True)
    var = jnp.mean((t - mu) ** 2, axis=-1, keepdims=True)
    y = (t - mu) * lax.rsqrt(var + 1e-5) * g_ref[...] + bt_ref[...]
    if apply_relu:
        y = jnp.maximum(y, 0.0)
    o_ref[...] = y + hin_ref[...]


def _post_call(apply_relu):
    return pl.pallas_call(
        functools.partial(_post_body, apply_relu=apply_relu),
        grid=(_NGRID,),
        in_specs=[
            pl.BlockSpec((_NB, D), lambda i: (i, 0)),
            pl.BlockSpec((_NB, D), lambda i: (i + _NGRID, 0)),
            pl.BlockSpec((_NB, D), lambda i: (i, 0)),
            pl.BlockSpec((_NB, D), lambda i: (i + _NGRID, 0)),
            pl.BlockSpec((_NB, D), lambda i: (i, 0)),
            pl.BlockSpec((1, D), lambda i: (0, 0)),
            pl.BlockSpec((1, D), lambda i: (0, 0)),
            pl.BlockSpec((1, D), lambda i: (0, 0)),
        ],
        out_specs=pl.BlockSpec((_NB, D), lambda i: (i, 0)),
        out_shape=jax.ShapeDtypeStruct((N, D), jnp.float32),
    )


def kernel(x, edge_index, edge_attr,
           Wn0, We0, A1w0, A1b0, A2w0, A2b0, b0, g0, bt0,
           Wn1, We1, A1w1, A1b1, A2w1, A2b1, b1, g1, bt1,
           Wn2, We2, A1w2, A1b2, A2w2, A2b2, b2, g2, bt2):
    src = edge_index[0]
    dst = edge_index[1]
    layers = [
        (Wn0, We0, A1w0, A1b0, A2w0, A2b0, b0, g0, bt0),
        (Wn1, We1, A1w1, A1b1, A2w1, A2b1, b1, g1, bt1),
        (Wn2, We2, A1w2, A1b2, A2w2, A2b2, b2, g2, bt2),
    ]
    catp = _np.concatenate([_np.arange(0, D, 2), _np.arange(1, D, 2)])
    inv = _np.empty(D, dtype=_np.int32)
    inv[0::2] = _np.arange(DH)
    inv[1::2] = _np.arange(DH, D)
    h = x
    for li, (Wn, We, A1w, A1b, A2w, A2b, b, g, bt) in enumerate(layers):
        a1_comb = jnp.concatenate([A1w[:D], A1w[D:]], axis=1)[:, catp]
        a1bias = jnp.concatenate([A1b, jnp.zeros((DH,), A1b.dtype)])[catp]
        wet_cat = We[:D][:, catp]
        t = _node_call(h, Wn, a1_comb, a1bias.reshape(1, D), wet_cat)
        a2 = A2w[:, 0]
        a2w_cat = jnp.concatenate([a2[0::2], a2[1::2]]).reshape(1, DH)
        web_cat = jnp.concatenate([We[D:][:, 0::2], We[D:][:, 1::2]], axis=1)
        sA, msA = _gather_A(t, src, dst)
        sB, msB = _gather_B(t, src, dst)
        msgA = _edge_call(sA, msA, edge_attr[:EH], a2w_cat,
                          A2b.reshape(1, 1), web_cat)
        msgB = _edge_call(sB, msB, edge_attr[EH:], a2w_cat,
                          A2b.reshape(1, 1), web_cat)
        partA = jnp.take(_scatter_A(msgA, dst), inv, axis=1)
        partB = jnp.take(_scatter_B(msgB, dst), inv, axis=1)
        h = _post_call(li < 2)(partA, partA, partB, partB, h,
                               b.reshape(1, D), g.reshape(1, D),
                               bt.reshape(1, D))
    return h


# speedup vs baseline: 1.2402x; 1.0453x over previous
import functools

import numpy as _np

import jax
import jax.numpy as jnp
from jax import lax
from jax.experimental import pallas as pl
from jax.experimental.pallas import tpu as pltpu
from jax.experimental.pallas import tpu_sc as plsc

N = 10000
E = 320000
D = 128
DH = 64

NC = 2
NS = 16
NW = NC * NS
EH = E // 2
EPW = EH // NW
C = 40
NCHUNK = EPW // C
CS = 40
NCHUNK_S = EPW // CS
RPS = N // NS

_mesh = plsc.VectorSubcoreMesh(core_axis_name="c", subcore_axis_name="s")


def _make_gather(hoff):
  @functools.partial(
    pl.kernel,
    mesh=_mesh,
    out_type=[
        jax.ShapeDtypeStruct((EH, D // 2), jnp.float32),
        jax.ShapeDtypeStruct((EH, D // 2), jnp.float32),
    ],
    scratch_types=[
        pltpu.VMEM((C,), jnp.int32),
        pltpu.VMEM((C,), jnp.int32),
        pltpu.VMEM((C,), jnp.int32),
        pltpu.VMEM((C,), jnp.int32),
        pltpu.VMEM((C, D), jnp.float32),
        pltpu.VMEM((C, D), jnp.float32),
        pltpu.VMEM((C, D), jnp.float32),
        pltpu.VMEM((C, D), jnp.float32),
        pltpu.VMEM((C, D // 2), jnp.float32),
        pltpu.VMEM((C, D // 2), jnp.float32),
        pltpu.SemaphoreType.DMA,
        pltpu.SemaphoreType.DMA,
        pltpu.SemaphoreType.DMA,
        pltpu.SemaphoreType.DMA,
        pltpu.SemaphoreType.DMA,
        pltpu.SemaphoreType.DMA,
        pltpu.SemaphoreType.DMA,
        pltpu.SemaphoreType.DMA,
        pltpu.SemaphoreType.DMA,
        pltpu.SemaphoreType.DMA,
        pltpu.SemaphoreType.DMA,
        pltpu.SemaphoreType.DMA,
    ],
  )
  def _gather_k(t_hbm, src_hbm, dst_hbm, s_out, ms_out,
              idxs0, idxs1, idxd0, idxd1, bufd0, bufd1, bufs0, bufs1,
              sbuf, msbuf,
              semd0, semd1, sems0, sems1, semi0, semi1, semj0, semj1,
              semw0, semw1, semv0, semv1):
    wid = lax.axis_index("s") * NC + lax.axis_index("c")
    obase = wid * EPW
    base = hoff + obase
    idxs = [idxs0, idxs1]
    idxd = [idxd0, idxd1]
    bufd = [bufd0, bufd1]
    bufs = [bufs0, bufs1]
    semd = [semd0, semd1]
    sems_ = [sems0, sems1]
    semi = [semi0, semi1]
    semj = [semj0, semj1]
    semw = [semw0, semw1]
    semv = [semv0, semv1]

    gd = [None, None]
    gs = [None, None]
    ih = [None, None]
    wh = [None, None]

    pltpu.sync_copy(src_hbm.at[pl.ds(base, C)], idxs[0])
    pltpu.sync_copy(dst_hbm.at[pl.ds(base, C)], idxd[0])
    gd[0] = pltpu.async_copy(t_hbm.at[idxd[0]], bufd[0], semd[0])
    gs[0] = pltpu.async_copy(t_hbm.at[idxs[0]], bufs[0], sems_[0])
    if NCHUNK > 1:
        ih[1] = (pltpu.async_copy(src_hbm.at[pl.ds(base + C, C)], idxs[1],
                                  semi[1]),
                 pltpu.async_copy(dst_hbm.at[pl.ds(base + C, C)], idxd[1],
                                  semj[1]))

    for k in range(NCHUNK):
        p = k % 2
        q = 1 - p
        off = base + k * C
        if k + 1 < NCHUNK:
            ih[q][0].wait()
            ih[q][1].wait()
            gd[q] = pltpu.async_copy(t_hbm.at[idxd[q]], bufd[q], semd[q])
            gs[q] = pltpu.async_copy(t_hbm.at[idxs[q]], bufs[q], sems_[q])
        gd[p].wait()
        gs[p].wait()
        if k + 2 < NCHUNK:
            off2 = off + 2 * C
            ih[p] = (pltpu.async_copy(src_hbm.at[pl.ds(off2, C)], idxs[p],
                                      semi[p]),
                     pltpu.async_copy(dst_hbm.at[pl.ds(off2, C)], idxd[p],
                                      semj[p]))
        if wh[0] is not None:
            for h in wh[0]:
                h.wait()
            wh[0] = None

        def addrow(r, c2, _p=p):
            for j in range(2):
                sbuf[r, pl.ds(j * 16, 16)] = bufd[_p][r, pl.ds(j * 16, 16)]
                sbuf[r, pl.ds(32 + j * 16, 16)] = bufs[_p][
                    r, pl.ds(32 + j * 16, 16)]
            for j in range(4):
                msbuf[r, pl.ds(j * 16, 16)] = bufs[_p][r,
                                                       pl.ds(64 + j * 16, 16)]
            return c2

        lax.fori_loop(0, C, addrow, 0)
        wh[0] = (
            pltpu.async_copy(sbuf, s_out.at[pl.ds(obase + k * C, C)],
                             semw[p]),
            pltpu.async_copy(msbuf, ms_out.at[pl.ds(obase + k * C, C)],
                             semv[p]),
        )

    if wh[0] is not None:
        for h in wh[0]:
            h.wait()

  return _gather_k


_gather_A = _make_gather(0)
_gather_B = _make_gather(EH)


def _make_scatter(hoff):
  @functools.partial(
    pl.kernel,
    mesh=_mesh,
    out_type=jax.ShapeDtypeStruct((2 * N, D), jnp.float32),
    scratch_types=[
        pltpu.VMEM((CS,), jnp.int32),
        pltpu.VMEM((CS,), jnp.int32),
        pltpu.VMEM((CS, D), jnp.float32),
        pltpu.VMEM((CS, D), jnp.float32),
        pltpu.VMEM_SHARED((N, D), jnp.float32),
        pltpu.SemaphoreType.DMA,
        pltpu.SemaphoreType.DMA,
        pltpu.SemaphoreType.DMA,
        pltpu.SemaphoreType.DMA,
    ],
  )
  def _scatter_k(msg_hbm, dst_hbm, out_hbm, idx0, idx1, buf0, buf1, acc,
               semm0, semm1, semi0, semi1):
    cid = lax.axis_index("c")
    sid = lax.axis_index("s")
    wid = sid * NC + cid
    idx = [idx0, idx1]
    buf = [buf0, buf1]
    semm = [semm0, semm1]
    semi = [semi0, semi1]

    def zrow(r, carry):
        for j in range(D // 16):
            buf0[r, pl.ds(j * 16, 16)] = jnp.zeros((16,), jnp.float32)
        return carry

    lax.fori_loop(0, CS, zrow, 0)
    nzc = N // CS
    for tt in range(-(-nzc // NS)):
        t = tt * NS + sid

        def zcopy(tv=t):
            pltpu.sync_copy(buf0, acc.at[pl.ds(tv * CS, CS)])

        pl.when(t < nzc)(zcopy)
    plsc.subcore_barrier()

    obase = wid * EPW
    gbase = hoff + obase
    lh = [None, None]
    lh[0] = (pltpu.async_copy(msg_hbm.at[pl.ds(obase, CS)], buf[0], semm[0]),
             pltpu.async_copy(dst_hbm.at[pl.ds(gbase, CS)], idx[0], semi[0]))
    for k in range(NCHUNK_S):
        p = k % 2
        q = 1 - p
        lh[p][0].wait()
        lh[p][1].wait()
        if k + 1 < NCHUNK_S:
            lh[q] = (pltpu.async_copy(
                         msg_hbm.at[pl.ds(obase + (k + 1) * CS, CS)],
                         buf[q], semm[q]),
                     pltpu.async_copy(
                         dst_hbm.at[pl.ds(gbase + (k + 1) * CS, CS)],
                         idx[q], semi[q]))
        pltpu.sync_copy(buf[p], acc.at[idx[p]], add=True)
    plsc.subcore_barrier()

    for tt in range(-(-nzc // NS)):
        t = tt * NS + sid

        def dcopy(tv=t):
            pltpu.sync_copy(acc.at[pl.ds(tv * CS, CS)],
                            out_hbm.at[pl.ds(cid * N + tv * CS, CS)])

        pl.when(t < nzc)(dcopy)

  return _scatter_k


_scatter_A = _make_scatter(0)
_scatter_B = _make_scatter(EH)


_NB = 400
_NGRID = N // _NB
_EB = 1600
_EGRID = EH // _EB


def _pack_words(x):
    u = lax.bitcast_convert_type(x.astype(jnp.bfloat16), jnp.uint16)
    lo = u[:, :DH].astype(jnp.uint32)
    hi = u[:, DH:].astype(jnp.uint32)
    return lax.bitcast_convert_type(lo | (hi << 16), jnp.float32)


def _node_body(h_ref, wn_ref, a1_ref, a1bias_ref, wet_ref, t_ref):
    hn = jnp.dot(h_ref[...], wn_ref[...], preferred_element_type=jnp.float32)
    p = (jnp.dot(hn, a1_ref[...], preferred_element_type=jnp.float32)
         + a1bias_ref[...])
    m = jnp.dot(hn, wet_ref[...], preferred_element_type=jnp.float32)
    t_ref[...] = jnp.concatenate([_pack_words(p), _pack_words(m)], axis=1)


_node_call = pl.pallas_call(
    _node_body,
    grid=(_NGRID,),
    in_specs=[
        pl.BlockSpec((_NB, D), lambda i: (i, 0)),
        pl.BlockSpec((D, D), lambda i: (0, 0)),
        pl.BlockSpec((D, D), lambda i: (0, 0)),
        pl.BlockSpec((1, D), lambda i: (0, 0)),
        pl.BlockSpec((D, D), lambda i: (0, 0)),
    ],
    out_specs=pl.BlockSpec((_NB, D), lambda i: (i, 0)),
    out_shape=jax.ShapeDtypeStruct((N, D), jnp.float32),
)


def _unpack_cat(words):
    u = lax.bitcast_convert_type(words, jnp.uint32)
    lo = lax.bitcast_convert_type(u << 16, jnp.float32)
    hi = lax.bitcast_convert_type(u & jnp.uint32(0xFFFF0000), jnp.float32)
    return jnp.concatenate([lo, hi], axis=-1)


def _edge_body(s_ref, ms_ref, ea_ref, a2w_ref, a2b_ref, web_ref, msg_ref):
    xw = s_ref[...]
    s = _unpack_cat(xw[:, :D // 4]) + _unpack_cat(xw[:, D // 4:])
    ms = _unpack_cat(ms_ref[...])
    srelu = jnp.maximum(s, 0.0)
    z = jnp.sum(srelu * a2w_ref[...], axis=-1, keepdims=True) + a2b_ref[0, 0]
    att = jax.nn.sigmoid(z)
    ec = (ea_ref[:, 0:1] * web_ref[0:1, :] + ea_ref[:, 1:2] * web_ref[1:2, :])
    msg_ref[...] = att * (ms + ec)


_edge_call = pl.pallas_call(
    _edge_body,
    grid=(_EGRID,),
    in_specs=[
        pl.BlockSpec((_EB, D // 2), lambda i: (i, 0)),
        pl.BlockSpec((_EB, D // 2), lambda i: (i, 0)),
        pl.BlockSpec((_EB, 2), lambda i: (i, 0)),
        pl.BlockSpec((1, DH), lambda i: (0, 0)),
        pl.BlockSpec((1, 1), lambda i: (0, 0)),
        pl.BlockSpec((2, D), lambda i: (0, 0)),
    ],
    out_specs=pl.BlockSpec((_EB, D), lambda i: (i, 0)),
    out_shape=jax.ShapeDtypeStruct((EH, D), jnp.float32),
)


def _post_body(p0_ref, p1_ref, p2_ref, p3_ref, hin_ref, b_ref, g_ref,
               bt_ref, o_ref, *, apply_relu):
    t = (p0_ref[...] + p1_ref[...] + p2_ref[...] + p3_ref[...]
         + b_ref[...])
    mu = jnp.mean(t, axis=-1, keepdims=True)
    var = jnp.mean((t - mu) ** 2, axis=-1, keepdims=True)
    y = (t - mu) * lax.rsqrt(var + 1e-5) * g_ref[...] + bt_ref[...]
    if apply_relu:
        y = jnp.maximum(y, 0.0)
    o_ref[...] = y + hin_ref[...]


def _post_call(apply_relu):
    return pl.pallas_call(
        functools.partial(_post_body, apply_relu=apply_relu),
        grid=(_NGRID,),
        in_specs=[
            pl.BlockSpec((_NB, D), lambda i: (i, 0)),
            pl.BlockSpec((_NB, D), lambda i: (i + _NGRID, 0)),
            pl.BlockSpec((_NB, D), lambda i: (i, 0)),
            pl.BlockSpec((_NB, D), lambda i: (i + _NGRID, 0)),
            pl.BlockSpec((_NB, D), lambda i: (i, 0)),
            pl.BlockSpec((1, D), lambda i: (0, 0)),
            pl.BlockSpec((1, D), lambda i: (0, 0)),
            pl.BlockSpec((1, D), lambda i: (0, 0)),
        ],
        out_specs=pl.BlockSpec((_NB, D), lambda i: (i, 0)),
        out_shape=jax.ShapeDtypeStruct((N, D), jnp.float32),
    )


def kernel(x, edge_index, edge_attr,
           Wn0, We0, A1w0, A1b0, A2w0, A2b0, b0, g0, bt0,
           Wn1, We1, A1w1, A1b1, A2w1, A2b1, b1, g1, bt1,
           Wn2, We2, A1w2, A1b2, A2w2, A2b2, b2, g2, bt2):
    src = edge_index[0]
    dst = edge_index[1]
    layers = [
        (Wn0, We0, A1w0, A1b0, A2w0, A2b0, b0, g0, bt0),
        (Wn1, We1, A1w1, A1b1, A2w1, A2b1, b1, g1, bt1),
        (Wn2, We2, A1w2, A1b2, A2w2, A2b2, b2, g2, bt2),
    ]
    catp = _np.concatenate([_np.arange(0, D, 2), _np.arange(1, D, 2)])
    inv = _np.empty(D, dtype=_np.int32)
    inv[0::2] = _np.arange(DH)
    inv[1::2] = _np.arange(DH, D)
    h = x[:, catp]
    for li, (Wn, We, A1w, A1b, A2w, A2b, b, g, bt) in enumerate(layers):
        a1_comb = jnp.concatenate([A1w[:D], A1w[D:]], axis=1)[:, catp]
        a1bias = jnp.concatenate([A1b, jnp.zeros((DH,), A1b.dtype)])[catp]
        wet_cat = We[:D][:, catp]
        wn_cat = Wn[catp, :]
        t = _node_call(h, wn_cat, a1_comb, a1bias.reshape(1, D), wet_cat)
        a2 = A2w[:, 0]
        a2w_cat = jnp.concatenate([a2[0::2], a2[1::2]]).reshape(1, DH)
        web_cat = jnp.concatenate([We[D:][:, 0::2], We[D:][:, 1::2]], axis=1)
        sA, msA = _gather_A(t, src, dst)
        sB, msB = _gather_B(t, src, dst)
        msgA = _edge_call(sA, msA, edge_attr[:EH], a2w_cat,
                          A2b.reshape(1, 1), web_cat)
        msgB = _edge_call(sB, msB, edge_attr[EH:], a2w_cat,
                          A2b.reshape(1, 1), web_cat)
        partA = _scatter_A(msgA, dst)
        partB = _scatter_B(msgB, dst)
        h = _post_call(li < 2)(partA, partA, partB, partB, h,
                               b[catp].reshape(1, D), g[catp].reshape(1, D),
                               bt[catp].reshape(1, D))
    return h[:, inv]
